# Initial kernel scaffold; baseline (speedup 1.0000x reference)
import functools

import jax
import jax.numpy as jnp
from jax import lax
from jax.experimental import pallas as pl
from jax.experimental.pallas import tpu as pltpu
from jax.experimental.pallas import tpu_sc as plsc

N = 10000
E = 320000
D = 128
ROWB = 1000
K = 80


def _expmap0(v):
    col = lax.broadcasted_iota(jnp.int32, v.shape, 1)
    u = jnp.where(col == 0, 0.0, v)
    n = jnp.sqrt(jnp.sum(u * u, axis=1, keepdims=True))
    n = jnp.maximum(n, 1e-6)
    en = jnp.exp(n)
    eni = 1.0 / en
    cosh_n = 0.5 * (en + eni)
    sinh_n = 0.5 * (en - eni)
    return jnp.where(col == 0, cosh_n, (sinh_n / n) * u)


def _logmap0(x):
    col = lax.broadcasted_iota(jnp.int32, x.shape, 1)
    time = jnp.maximum(x[:, 0:1], 1.0 + 1e-6)
    space = jnp.where(col == 0, 0.0, x)
    d = jnp.log(time + jnp.sqrt(time * time - 1.0))
    sn = jnp.sqrt(jnp.sum(space * space, axis=1, keepdims=True))
    sn = jnp.maximum(sn, 1e-6)
    return (d / sn) * space


def _sphere_proj(v):
    n = jnp.sqrt(jnp.sum(v * v, axis=1, keepdims=True))
    return v / jnp.maximum(n, 1e-6)


def _dot(a, b):
    return jnp.dot(a, b, preferred_element_type=jnp.float32)


def _enc_body(x_ref, w1, b1, w2, b2, wh, bh, ws, bs, wh0, bh0, ws0, bs0,
              xe_out, mh_out, ms_out):
    x = x_ref[...]
    h = jnp.maximum(_dot(x, w1[...]) + b1[...], 0.0)
    xe_out[...] = _dot(h, w2[...]) + b2[...]
    xh = _expmap0(_dot(x, wh[...]) + bh[...])
    t = _logmap0(xh)
    mh_out[...] = _dot(t, wh0[...]) + bh0[...]
    xs = _sphere_proj(_dot(x, ws[...]) + bs[...])
    ms_out[...] = _dot(xs, ws0[...]) + bs0[...]


def _mid_body(sh_ref, ss_ref, cnt_ref, wh1, bh1, ws1, bs1, mh_out, ms_out):
    cnt = jnp.maximum(cnt_ref[:, 0:1], 1.0)
    xh = _expmap0(sh_ref[...] / cnt)
    t = _logmap0(xh)
    mh_out[...] = _dot(t, wh1[...]) + bh1[...]
    xs = _sphere_proj(ss_ref[...] / cnt)
    ms_out[...] = _dot(xs, ws1[...]) + bs1[...]


def _fin_body(sh_ref, ss_ref, cnt_ref, xh_out, xs_out):
    cnt = jnp.maximum(cnt_ref[:, 0:1], 1.0)
    xh_out[...] = _expmap0(sh_ref[...] / cnt)
    xs_out[...] = _sphere_proj(ss_ref[...] / cnt)


_ROW_SPEC = pl.BlockSpec((ROWB, D), lambda i: (i, 0))
_CNT_SPEC = pl.BlockSpec((ROWB, 16), lambda i: (i, 0))
_W_SPEC = pl.BlockSpec((D, D), lambda i: (0, 0))
_B_SPEC = pl.BlockSpec((1, D), lambda i: (0, 0))
_GRID = (N // ROWB,)
_F32 = functools.partial(jax.ShapeDtypeStruct, dtype=jnp.float32)


def _tc_encoder(x, W1, b1, W2, b2, Wh, bh, Ws, bs, WH0, bH0, WS0, bS0):
    return pl.pallas_call(
        _enc_body,
        grid=_GRID,
        in_specs=[_ROW_SPEC] + [_W_SPEC, _B_SPEC] * 6,
        out_specs=[_ROW_SPEC] * 3,
        out_shape=[_F32((N, D))] * 3,
    )(x, W1, b1, W2, b2, Wh, bh, Ws, bs, WH0, bH0, WS0, bS0)


def _tc_mid(sumH, sumS, cnt16, WH1, bH1, WS1, bS1):
    return pl.pallas_call(
        _mid_body,
        grid=_GRID,
        in_specs=[_ROW_SPEC, _ROW_SPEC, _CNT_SPEC] + [_W_SPEC, _B_SPEC] * 2,
        out_specs=[_ROW_SPEC] * 2,
        out_shape=[_F32((N, D))] * 2,
    )(sumH, sumS, cnt16, WH1, bH1, WS1, bS1)


def _tc_final(sumH, sumS, cnt16):
    return pl.pallas_call(
        _fin_body,
        grid=_GRID,
        in_specs=[_ROW_SPEC, _ROW_SPEC, _CNT_SPEC],
        out_specs=[_ROW_SPEC] * 2,
        out_shape=[_F32((N, D))] * 2,
    )(sumH, sumS, cnt16)


def _sc_agg(mh, ms, src, dst, with_counts):
    info = plsc.get_sparse_core_info()
    NS = info.num_subcores
    EPT = E // NS
    NCH = EPT // K
    RPT = ((N + NS - 1) // NS + K - 1) // K * K
    mesh = plsc.VectorSubcoreMesh(core_axis_name="c", subcore_axis_name="s")

    out_type = [_F32((2, N, D))]
    scratch = [
        pltpu.VMEM_SHARED((N, D), jnp.float32),
        pltpu.VMEM((K,), jnp.int32),
        pltpu.VMEM((K,), jnp.int32),
        pltpu.VMEM((K, D), jnp.float32),
        pltpu.VMEM((K, D), jnp.float32),
        pltpu.SemaphoreType.DMA,
    ]
    if with_counts:
        out_type.append(_F32((N, 16)))
        scratch += [
            pltpu.VMEM_SHARED((N, 16), jnp.float32),
            pltpu.VMEM((K, 16), jnp.float32),
            pltpu.VMEM((K, 16), jnp.float32),
        ]

    def body(mh_hbm, ms_hbm, src_hbm, dst_hbm, out_hbm, *rest):
        if with_counts:
            (cnt_hbm, acc_sh, src_v, dst_v, rows_v, zrows_v, sem,
             cnt_sh, ones_v, z16_v) = rest
        else:
            acc_sh, src_v, dst_v, rows_v, zrows_v, sem = rest
        cid = lax.axis_index("c")
        sid = lax.axis_index("s")

        def fill(r, _):
            for l in range(D // 16):
                zrows_v[r, pl.ds(l * 16, 16)] = jnp.zeros((16,), jnp.float32)
            if with_counts:
                ones_v[r, pl.ds(0, 16)] = jnp.ones((16,), jnp.float32)
                z16_v[r, pl.ds(0, 16)] = jnp.zeros((16,), jnp.float32)
            return 0
        lax.fori_loop(0, K, fill, 0)

        row0 = sid * RPT
        nz = jnp.minimum(jnp.maximum(N - row0, 0), RPT) // K
        def zb(j, _):
            pltpu.sync_copy(zrows_v, acc_sh.at[pl.ds(row0 + j * K, K)])
            if with_counts:
                @pl.when(cid == 0)
                def _():
                    pltpu.sync_copy(z16_v, cnt_sh.at[pl.ds(row0 + j * K, K)])
            return 0
        lax.fori_loop(0, nz, zb, 0)
        plsc.subcore_barrier()

        ebase = sid * EPT
        def mb(j, _):
            b = ebase + j * K
            pltpu.sync_copy(src_hbm.at[pl.ds(b, K)], src_v)
            pltpu.sync_copy(dst_hbm.at[pl.ds(b, K)], dst_v)
            @pl.when(cid == 0)
            def _():
                pltpu.async_copy(mh_hbm.at[src_v], rows_v, sem).wait()
            @pl.when(cid == 1)
            def _():
                pltpu.async_copy(ms_hbm.at[src_v], rows_v, sem).wait()
            pltpu.sync_copy(rows_v, acc_sh.at[dst_v], add=True)
            if with_counts:
                @pl.when(cid == 0)
                def _():
                    pltpu.sync_copy(ones_v, cnt_sh.at[dst_v], add=True)
            return 0
        lax.fori_loop(0, NCH, mb, 0)
        plsc.subcore_barrier()

        def ob(j, _):
            r = row0 + j * K
            pltpu.sync_copy(acc_sh.at[pl.ds(r, K)], out_hbm.at[cid, pl.ds(r, K)])
            if with_counts:
                @pl.when(cid == 0)
                def _():
                    pltpu.sync_copy(cnt_sh.at[pl.ds(r, K)], cnt_hbm.at[pl.ds(r, K)])
            return 0
        lax.fori_loop(0, nz, ob, 0)

    call = pl.kernel(body, out_type=out_type, mesh=mesh, scratch_types=scratch)
    return call(mh, ms, src, dst)


def kernel(x, edge_index, W1, b1, W2, b2, Wh, bh, Ws, bs, WH, bH, WS, bS):
    src = edge_index[0].astype(jnp.int32)
    dst = edge_index[1].astype(jnp.int32)
    r1 = lambda v: v.reshape(1, D)

    x_E, mH, mS = _tc_encoder(
        x, W1, r1(b1), W2, r1(b2), Wh, r1(bh), Ws, r1(bs),
        WH[0], r1(bH[0]), WS[0], r1(bS[0]))

    sums, cnt16 = _sc_agg(mH, mS, src, dst, with_counts=True)
    mH, mS = _tc_mid(sums[0], sums[1], cnt16, WH[1], r1(bH[1]), WS[1], r1(bS[1]))

    sums2 = _sc_agg(mH, mS, src, dst, with_counts=False)
    if isinstance(sums2, (list, tuple)):
        sums2 = sums2[0]
    x_H, x_S = _tc_final(sums2[0], sums2[1], cnt16)
    return (x_E, x_H, x_S)


# speedup vs baseline: 2.9940x; 2.9940x over previous
import functools

import jax
import jax.numpy as jnp
from jax import lax
from jax.experimental import pallas as pl
from jax.experimental.pallas import tpu as pltpu
from jax.experimental.pallas import tpu_sc as plsc

N = 10000
E = 320000
D = 128
ROWB = 1000
K = 80
NP = 10240
RZ = 80


def _expmap0(v):
    col = lax.broadcasted_iota(jnp.int32, v.shape, 1)
    u = jnp.where(col == 0, 0.0, v)
    n = jnp.sqrt(jnp.sum(u * u, axis=1, keepdims=True))
    n = jnp.maximum(n, 1e-6)
    en = jnp.exp(n)
    eni = 1.0 / en
    cosh_n = 0.5 * (en + eni)
    sinh_n = 0.5 * (en - eni)
    return jnp.where(col == 0, cosh_n, (sinh_n / n) * u)


def _logmap0(x):
    col = lax.broadcasted_iota(jnp.int32, x.shape, 1)
    time = jnp.maximum(x[:, 0:1], 1.0 + 1e-6)
    space = jnp.where(col == 0, 0.0, x)
    d = jnp.log(time + jnp.sqrt(time * time - 1.0))
    sn = jnp.sqrt(jnp.sum(space * space, axis=1, keepdims=True))
    sn = jnp.maximum(sn, 1e-6)
    return (d / sn) * space


def _sphere_proj(v):
    n = jnp.sqrt(jnp.sum(v * v, axis=1, keepdims=True))
    return v / jnp.maximum(n, 1e-6)


def _dot(a, b):
    return jnp.dot(a, b, preferred_element_type=jnp.float32)


def _enc_body(x_ref, w1, b1, w2, b2, wh, bh, ws, bs, wh0, bh0, ws0, bs0,
              xe_out, m2_out):
    x = x_ref[...]
    h = jnp.maximum(_dot(x, w1[...]) + b1[...], 0.0)
    xe_out[...] = _dot(h, w2[...]) + b2[...]
    xh = _expmap0(_dot(x, wh[...]) + bh[...])
    t = _logmap0(xh)
    m2_out[0] = _dot(t, wh0[...]) + bh0[...]
    xs = _sphere_proj(_dot(x, ws[...]) + bs[...])
    m2_out[1] = _dot(xs, ws0[...]) + bs0[...]


def _mid_body(sh_ref, ss_ref, ca_ref, cb_ref, wh1, bh1, ws1, bs1, m2_out):
    cnt = jnp.maximum(ca_ref[:, 0:1] + cb_ref[:, 0:1], 1.0)
    xh = _expmap0(sh_ref[...] / cnt)
    t = _logmap0(xh)
    m2_out[0] = _dot(t, wh1[...]) + bh1[...]
    xs = _sphere_proj(ss_ref[...] / cnt)
    m2_out[1] = _dot(xs, ws1[...]) + bs1[...]


def _fin_body(sh_ref, ss_ref, ca_ref, cb_ref, xh_out, xs_out):
    cnt = jnp.maximum(ca_ref[:, 0:1] + cb_ref[:, 0:1], 1.0)
    xh_out[...] = _expmap0(sh_ref[...] / cnt)
    xs_out[...] = _sphere_proj(ss_ref[...] / cnt)


_ROW_SPEC = pl.BlockSpec((ROWB, D), lambda i: (i, 0))
_M2_SPEC = pl.BlockSpec((2, ROWB, D), lambda i: (0, i, 0))
_W_SPEC = pl.BlockSpec((D, D), lambda i: (0, 0))
_B_SPEC = pl.BlockSpec((1, D), lambda i: (0, 0))
_GRID = (N // ROWB,)
_F32 = functools.partial(jax.ShapeDtypeStruct, dtype=jnp.float32)


def _tc_encoder(x, W1, b1, W2, b2, Wh, bh, Ws, bs, WH0, bH0, WS0, bS0):
    return pl.pallas_call(
        _enc_body,
        grid=_GRID,
        in_specs=[_ROW_SPEC] + [_W_SPEC, _B_SPEC] * 6,
        out_specs=[_ROW_SPEC, _M2_SPEC],
        out_shape=[_F32((N, D)), _F32((2, N, D))],
    )(x, W1, b1, W2, b2, Wh, bh, Ws, bs, WH0, bH0, WS0, bS0)


def _tc_mid(sumH, sumS, cnt_a, cnt_b, WH1, bH1, WS1, bS1):
    return pl.pallas_call(
        _mid_body,
        grid=_GRID,
        in_specs=[_ROW_SPEC] * 4 + [_W_SPEC, _B_SPEC] * 2,
        out_specs=[_M2_SPEC],
        out_shape=[_F32((2, N, D))],
    )(sumH, sumS, cnt_a, cnt_b, WH1, bH1, WS1, bS1)


def _tc_final(sumH, sumS, cnt_a, cnt_b):
    return pl.pallas_call(
        _fin_body,
        grid=_GRID,
        in_specs=[_ROW_SPEC] * 4,
        out_specs=[_ROW_SPEC] * 2,
        out_shape=[_F32((N, D))] * 2,
    )(sumH, sumS, cnt_a, cnt_b)


_MESH = dict(core_axis_name="c", subcore_axis_name="s")


def _zero_acc(acc_sh, zb_v, row0):
    for j in range(NP // 16 // RZ):
        pltpu.sync_copy(zb_v, acc_sh.at[pl.ds(row0 + j * RZ, RZ)])


def _copy_out(acc_sh, out_hbm, row0, roff):
    for j in range(NP // 16 // RZ):
        r = row0 + j * RZ
        pltpu.sync_copy(acc_sh.at[pl.ds(r, RZ)], out_hbm.at[pl.ds(roff + r, RZ)])


def _sc_agg(m2, src, dst, zeros):
    info = plsc.get_sparse_core_info()
    NS = info.num_subcores
    EPT = E // NS
    NCH = EPT // K

    def body(m2_hbm, src_hbm, dst_hbm, zeros_hbm, out_hbm,
             acc_sh, src_v, dst_v, rows_v, zb_v, sem):
        cid = lax.axis_index("c")
        sid = lax.axis_index("s")
        row0 = sid * (NP // NS)

        pltpu.sync_copy(zeros_hbm, zb_v)
        _zero_acc(acc_sh, zb_v, row0)
        plsc.subcore_barrier()

        ebase = sid * EPT
        def mb(j, _):
            b = ebase + j * K
            pltpu.sync_copy(src_hbm.at[pl.ds(cid * E + b, K)], src_v)
            pltpu.sync_copy(dst_hbm.at[pl.ds(b, K)], dst_v)
            pltpu.async_copy(m2_hbm.at[src_v], rows_v, sem).wait()
            pltpu.sync_copy(rows_v, acc_sh.at[dst_v], add=True)
            return 0
        lax.fori_loop(0, NCH, mb, 0)
        plsc.subcore_barrier()
        _copy_out(acc_sh, out_hbm, row0, cid * NP)

    call = pl.kernel(
        body,
        out_type=_F32((2 * NP, D)),
        mesh=plsc.VectorSubcoreMesh(**_MESH),
        scratch_types=[
            pltpu.VMEM_SHARED((NP, D), jnp.float32),
            pltpu.VMEM((K,), jnp.int32),
            pltpu.VMEM((K,), jnp.int32),
            pltpu.VMEM((K, D), jnp.float32),
            pltpu.VMEM((RZ, D), jnp.float32),
            pltpu.SemaphoreType.DMA,
        ])
    return call(m2, src, dst, zeros)


def _sc_counts(dst, zeros, ones):
    info = plsc.get_sparse_core_info()
    NS = info.num_subcores
    EPC = E // 2
    EPT = EPC // NS
    NCH = EPT // K

    def body(dst_hbm, zeros_hbm, ones_hbm, out_hbm,
             acc_sh, dst_v, ones_v, zb_v):
        cid = lax.axis_index("c")
        sid = lax.axis_index("s")
        row0 = sid * (NP // NS)

        pltpu.sync_copy(zeros_hbm, zb_v)
        pltpu.sync_copy(ones_hbm, ones_v)
        _zero_acc(acc_sh, zb_v, row0)
        plsc.subcore_barrier()

        ebase = cid * EPC + sid * EPT
        def mb(j, _):
            pltpu.sync_copy(dst_hbm.at[pl.ds(ebase + j * K, K)], dst_v)
            pltpu.sync_copy(ones_v, acc_sh.at[dst_v], add=True)
            return 0
        lax.fori_loop(0, NCH, mb, 0)
        plsc.subcore_barrier()
        _copy_out(acc_sh, out_hbm, row0, cid * NP)

    call = pl.kernel(
        body,
        out_type=_F32((2 * NP, D)),
        mesh=plsc.VectorSubcoreMesh(**_MESH),
        scratch_types=[
            pltpu.VMEM_SHARED((NP, D), jnp.float32),
            pltpu.VMEM((K,), jnp.int32),
            pltpu.VMEM((K, D), jnp.float32),
            pltpu.VMEM((RZ, D), jnp.float32),
        ])
    return call(dst, zeros, ones)


def kernel(x, edge_index, W1, b1, W2, b2, Wh, bh, Ws, bs, WH, bH, WS, bS):
    src = edge_index[0].astype(jnp.int32)
    dst = edge_index[1].astype(jnp.int32)
    src = jnp.concatenate([src, src + N])
    zeros = jnp.zeros((RZ, D), jnp.float32)
    ones = jnp.ones((K, D), jnp.float32)
    r1 = lambda v: v.reshape(1, D)

    cnt = _sc_counts(dst, zeros, ones)
    cnt_a, cnt_b = cnt[:N], cnt[NP:NP + N]

    x_E, m2 = _tc_encoder(
        x, W1, r1(b1), W2, r1(b2), Wh, r1(bh), Ws, r1(bs),
        WH[0], r1(bH[0]), WS[0], r1(bS[0]))

    sums = _sc_agg(m2.reshape(2 * N, D), src, dst, zeros)
    (m2,) = _tc_mid(sums[:N], sums[NP:NP + N], cnt_a, cnt_b,
                    WH[1], r1(bH[1]), WS[1], r1(bS[1]))

    sums2 = _sc_agg(m2.reshape(2 * N, D), src, dst, zeros)
    x_H, x_S = _tc_final(sums2[:N], sums2[NP:NP + N], cnt_a, cnt_b)
    return (x_E, x_H, x_S)


# speedup vs baseline: 4.6111x; 1.5401x over previous
import functools

import jax
import jax.numpy as jnp
from jax import lax
from jax.experimental import pallas as pl
from jax.experimental.pallas import tpu as pltpu
from jax.experimental.pallas import tpu_sc as plsc

N = 10000
E = 320000
D = 128
ROWB = 1000
K = 80
NP = 10240
RZ = 80


def _expmap0(v):
    col = lax.broadcasted_iota(jnp.int32, v.shape, 1)
    u = jnp.where(col == 0, 0.0, v)
    n = jnp.sqrt(jnp.sum(u * u, axis=1, keepdims=True))
    n = jnp.maximum(n, 1e-6)
    en = jnp.exp(n)
    eni = 1.0 / en
    cosh_n = 0.5 * (en + eni)
    sinh_n = 0.5 * (en - eni)
    return jnp.where(col == 0, cosh_n, (sinh_n / n) * u)


def _logmap0(x):
    col = lax.broadcasted_iota(jnp.int32, x.shape, 1)
    time = jnp.maximum(x[:, 0:1], 1.0 + 1e-6)
    space = jnp.where(col == 0, 0.0, x)
    d = jnp.log(time + jnp.sqrt(time * time - 1.0))
    sn = jnp.sqrt(jnp.sum(space * space, axis=1, keepdims=True))
    sn = jnp.maximum(sn, 1e-6)
    return (d / sn) * space


def _sphere_proj(v):
    n = jnp.sqrt(jnp.sum(v * v, axis=1, keepdims=True))
    return v / jnp.maximum(n, 1e-6)


def _dot(a, b):
    return jnp.dot(a, b, preferred_element_type=jnp.float32)


def _enc_body(x_ref, w1, b1, w2, b2, wh, bh, ws, bs, wh0, bh0, ws0, bs0,
              xe_out, m2_out):
    x = x_ref[...]
    h = jnp.maximum(_dot(x, w1[...]) + b1[...], 0.0)
    xe_out[...] = _dot(h, w2[...]) + b2[...]
    xh = _expmap0(_dot(x, wh[...]) + bh[...])
    t = _logmap0(xh)
    m2_out[0] = _dot(t, wh0[...]) + bh0[...]
    xs = _sphere_proj(_dot(x, ws[...]) + bs[...])
    m2_out[1] = _dot(xs, ws0[...]) + bs0[...]


def _mid_body(sh_ref, ss_ref, ca_ref, cb_ref, wh1, bh1, ws1, bs1, m2_out):
    cnt = jnp.maximum(ca_ref[:, 0:1] + cb_ref[:, 0:1], 1.0)
    xh = _expmap0(sh_ref[...] / cnt)
    t = _logmap0(xh)
    m2_out[0] = _dot(t, wh1[...]) + bh1[...]
    xs = _sphere_proj(ss_ref[...] / cnt)
    m2_out[1] = _dot(xs, ws1[...]) + bs1[...]


def _fin_body(sh_ref, ss_ref, ca_ref, cb_ref, xh_out, xs_out):
    cnt = jnp.maximum(ca_ref[:, 0:1] + cb_ref[:, 0:1], 1.0)
    xh_out[...] = _expmap0(sh_ref[...] / cnt)
    xs_out[...] = _sphere_proj(ss_ref[...] / cnt)


_ROW_SPEC = pl.BlockSpec((ROWB, D), lambda i: (i, 0))
_M2_SPEC = pl.BlockSpec((2, ROWB, D), lambda i: (0, i, 0))
_W_SPEC = pl.BlockSpec((D, D), lambda i: (0, 0))
_B_SPEC = pl.BlockSpec((1, D), lambda i: (0, 0))
_GRID = (N // ROWB,)
_F32 = functools.partial(jax.ShapeDtypeStruct, dtype=jnp.float32)


def _tc_encoder(x, W1, b1, W2, b2, Wh, bh, Ws, bs, WH0, bH0, WS0, bS0):
    return pl.pallas_call(
        _enc_body,
        grid=_GRID,
        in_specs=[_ROW_SPEC] + [_W_SPEC, _B_SPEC] * 6,
        out_specs=[_ROW_SPEC, _M2_SPEC],
        out_shape=[_F32((N, D)), _F32((2, N, D))],
    )(x, W1, b1, W2, b2, Wh, bh, Ws, bs, WH0, bH0, WS0, bS0)


def _tc_mid(sumH, sumS, cnt_a, cnt_b, WH1, bH1, WS1, bS1):
    return pl.pallas_call(
        _mid_body,
        grid=_GRID,
        in_specs=[_ROW_SPEC] * 4 + [_W_SPEC, _B_SPEC] * 2,
        out_specs=[_M2_SPEC],
        out_shape=[_F32((2, N, D))],
    )(sumH, sumS, cnt_a, cnt_b, WH1, bH1, WS1, bS1)


def _tc_final(sumH, sumS, cnt_a, cnt_b):
    return pl.pallas_call(
        _fin_body,
        grid=_GRID,
        in_specs=[_ROW_SPEC] * 4,
        out_specs=[_ROW_SPEC] * 2,
        out_shape=[_F32((N, D))] * 2,
    )(sumH, sumS, cnt_a, cnt_b)


_MESH = dict(core_axis_name="c", subcore_axis_name="s")


def _zero_acc(acc_sh, zb_v, row0):
    for j in range(NP // 16 // RZ):
        pltpu.sync_copy(zb_v, acc_sh.at[pl.ds(row0 + j * RZ, RZ)])


def _copy_out(acc_sh, out_hbm, row0, roff):
    for j in range(NP // 16 // RZ):
        r = row0 + j * RZ
        pltpu.sync_copy(acc_sh.at[pl.ds(r, RZ)], out_hbm.at[pl.ds(roff + r, RZ)])


def _sc_agg(m2, src, dst, zeros):
    info = plsc.get_sparse_core_info()
    NS = info.num_subcores
    EPT = E // NS
    NCH = EPT // K

    def body(m2_hbm, src_hbm, dst_hbm, zeros_hbm, out_hbm,
             acc_sh, src_a, dst_a, rows_a, src_b, dst_b, rows_b,
             zb_v, sem_a, sem_b):
        cid = lax.axis_index("c")
        sid = lax.axis_index("s")
        row0 = sid * (NP // NS)

        pltpu.sync_copy(zeros_hbm, zb_v)
        _zero_acc(acc_sh, zb_v, row0)
        plsc.subcore_barrier()

        ebase = sid * EPT

        def load(j, src_v, dst_v):
            b = ebase + j * K
            pltpu.sync_copy(src_hbm.at[pl.ds(cid * E + b, K)], src_v)
            pltpu.sync_copy(dst_hbm.at[pl.ds(b, K)], dst_v)

        def start(src_v, rows_v, sem):
            pltpu.async_copy(m2_hbm.at[src_v], rows_v, sem)

        def drain(dst_v, rows_v, sem):
            pltpu.make_async_copy(zeros_hbm, rows_v, sem).wait()
            pltpu.sync_copy(rows_v, acc_sh.at[dst_v], add=True)

        load(0, src_a, dst_a)
        start(src_a, rows_a, sem_a)

        def mb(jj, _):
            j = 2 * jj
            load(j + 1, src_b, dst_b)
            start(src_b, rows_b, sem_b)
            drain(dst_a, rows_a, sem_a)
            load(j + 2, src_a, dst_a)
            start(src_a, rows_a, sem_a)
            drain(dst_b, rows_b, sem_b)
            return 0
        lax.fori_loop(0, NCH // 2 - 1, mb, 0)

        load(NCH - 1, src_b, dst_b)
        start(src_b, rows_b, sem_b)
        drain(dst_a, rows_a, sem_a)
        drain(dst_b, rows_b, sem_b)

        plsc.subcore_barrier()
        _copy_out(acc_sh, out_hbm, row0, cid * NP)

    call = pl.kernel(
        body,
        out_type=_F32((2 * NP, D)),
        mesh=plsc.VectorSubcoreMesh(**_MESH),
        scratch_types=[
            pltpu.VMEM_SHARED((NP, D), jnp.float32),
            pltpu.VMEM((K,), jnp.int32),
            pltpu.VMEM((K,), jnp.int32),
            pltpu.VMEM((K, D), jnp.float32),
            pltpu.VMEM((K,), jnp.int32),
            pltpu.VMEM((K,), jnp.int32),
            pltpu.VMEM((K, D), jnp.float32),
            pltpu.VMEM((RZ, D), jnp.float32),
            pltpu.SemaphoreType.DMA,
            pltpu.SemaphoreType.DMA,
        ])
    return call(m2, src, dst, zeros)


def _sc_counts(dst, zeros, ones):
    info = plsc.get_sparse_core_info()
    NS = info.num_subcores
    EPC = E // 2
    EPT = EPC // NS
    NCH = EPT // K

    def body(dst_hbm, zeros_hbm, ones_hbm, out_hbm,
             acc_sh, dst_v, ones_v, zb_v):
        cid = lax.axis_index("c")
        sid = lax.axis_index("s")
        row0 = sid * (NP // NS)

        pltpu.sync_copy(zeros_hbm, zb_v)
        pltpu.sync_copy(ones_hbm, ones_v)
        _zero_acc(acc_sh, zb_v, row0)
        plsc.subcore_barrier()

        ebase = cid * EPC + sid * EPT
        def mb(j, _):
            pltpu.sync_copy(dst_hbm.at[pl.ds(ebase + j * K, K)], dst_v)
            pltpu.sync_copy(ones_v, acc_sh.at[dst_v], add=True)
            return 0
        lax.fori_loop(0, NCH, mb, 0)
        plsc.subcore_barrier()
        _copy_out(acc_sh, out_hbm, row0, cid * NP)

    call = pl.kernel(
        body,
        out_type=_F32((2 * NP, D)),
        mesh=plsc.VectorSubcoreMesh(**_MESH),
        scratch_types=[
            pltpu.VMEM_SHARED((NP, D), jnp.float32),
            pltpu.VMEM((K,), jnp.int32),
            pltpu.VMEM((K, D), jnp.float32),
            pltpu.VMEM((RZ, D), jnp.float32),
        ])
    return call(dst, zeros, ones)


def kernel(x, edge_index, W1, b1, W2, b2, Wh, bh, Ws, bs, WH, bH, WS, bS):
    src = edge_index[0].astype(jnp.int32)
    dst = edge_index[1].astype(jnp.int32)
    src = jnp.concatenate([src, src + N])
    zeros = jnp.zeros((RZ, D), jnp.float32)
    ones = jnp.ones((K, D), jnp.float32)
    r1 = lambda v: v.reshape(1, D)

    cnt = _sc_counts(dst, zeros, ones)
    cnt_a, cnt_b = cnt[:N], cnt[NP:NP + N]

    x_E, m2 = _tc_encoder(
        x, W1, r1(b1), W2, r1(b2), Wh, r1(bh), Ws, r1(bs),
        WH[0], r1(bH[0]), WS[0], r1(bS[0]))

    sums = _sc_agg(m2.reshape(2 * N, D), src, dst, zeros)
    (m2,) = _tc_mid(sums[:N], sums[NP:NP + N], cnt_a, cnt_b,
                    WH[1], r1(bH[1]), WS[1], r1(bS[1]))

    sums2 = _sc_agg(m2.reshape(2 * N, D), src, dst, zeros)
    x_H, x_S = _tc_final(sums2[:N], sums2[NP:NP + N], cnt_a, cnt_b)
    return (x_E, x_H, x_S)


# speedup vs baseline: 4.9964x; 1.0836x over previous
import functools

import jax
import jax.numpy as jnp
from jax import lax
from jax.experimental import pallas as pl
from jax.experimental.pallas import tpu as pltpu
from jax.experimental.pallas import tpu_sc as plsc

N = 10000
E = 320000
D = 128
ROWB = 1000
K = 80
NP = 10240
RZ = 80


def _expmap0(v):
    col = lax.broadcasted_iota(jnp.int32, v.shape, 1)
    u = jnp.where(col == 0, 0.0, v)
    n = jnp.sqrt(jnp.sum(u * u, axis=1, keepdims=True))
    n = jnp.maximum(n, 1e-6)
    en = jnp.exp(n)
    eni = 1.0 / en
    cosh_n = 0.5 * (en + eni)
    sinh_n = 0.5 * (en - eni)
    return jnp.where(col == 0, cosh_n, (sinh_n / n) * u)


def _logmap0(x):
    col = lax.broadcasted_iota(jnp.int32, x.shape, 1)
    time = jnp.maximum(x[:, 0:1], 1.0 + 1e-6)
    space = jnp.where(col == 0, 0.0, x)
    d = jnp.log(time + jnp.sqrt(time * time - 1.0))
    sn = jnp.sqrt(jnp.sum(space * space, axis=1, keepdims=True))
    sn = jnp.maximum(sn, 1e-6)
    return (d / sn) * space


def _sphere_proj(v):
    n = jnp.sqrt(jnp.sum(v * v, axis=1, keepdims=True))
    return v / jnp.maximum(n, 1e-6)


def _dot(a, b):
    return jnp.dot(a, b, preferred_element_type=jnp.float32)


def _enc_body(x_ref, w1, b1, w2, b2, wh, bh, ws, bs, wh0, bh0, ws0, bs0,
              xe_out, m2_out):
    x = x_ref[...]
    h = jnp.maximum(_dot(x, w1[...]) + b1[...], 0.0)
    xe_out[...] = _dot(h, w2[...]) + b2[...]
    xh = _expmap0(_dot(x, wh[...]) + bh[...])
    t = _logmap0(xh)
    m2_out[0] = _dot(t, wh0[...]) + bh0[...]
    xs = _sphere_proj(_dot(x, ws[...]) + bs[...])
    m2_out[1] = _dot(xs, ws0[...]) + bs0[...]


def _mid_body(sh_ref, ss_ref, ca_ref, cb_ref, wh1, bh1, ws1, bs1, m2_out):
    cnt = jnp.maximum(ca_ref[:, 0:1] + cb_ref[:, 0:1], 1.0)
    xh = _expmap0(sh_ref[...] / cnt)
    t = _logmap0(xh)
    m2_out[0] = _dot(t, wh1[...]) + bh1[...]
    xs = _sphere_proj(ss_ref[...] / cnt)
    m2_out[1] = _dot(xs, ws1[...]) + bs1[...]


def _fin_body(sh_ref, ss_ref, ca_ref, cb_ref, xh_out, xs_out):
    cnt = jnp.maximum(ca_ref[:, 0:1] + cb_ref[:, 0:1], 1.0)
    xh_out[...] = _expmap0(sh_ref[...] / cnt)
    xs_out[...] = _sphere_proj(ss_ref[...] / cnt)


_ROW_SPEC = pl.BlockSpec((ROWB, D), lambda i: (i, 0))
_M2_SPEC = pl.BlockSpec((2, ROWB, D), lambda i: (0, i, 0))
_W_SPEC = pl.BlockSpec((D, D), lambda i: (0, 0))
_B_SPEC = pl.BlockSpec((1, D), lambda i: (0, 0))
_GRID = (N // ROWB,)
_F32 = functools.partial(jax.ShapeDtypeStruct, dtype=jnp.float32)


def _tc_encoder(x, W1, b1, W2, b2, Wh, bh, Ws, bs, WH0, bH0, WS0, bS0):
    return pl.pallas_call(
        _enc_body,
        grid=_GRID,
        in_specs=[_ROW_SPEC] + [_W_SPEC, _B_SPEC] * 6,
        out_specs=[_ROW_SPEC, _M2_SPEC],
        out_shape=[_F32((N, D)), _F32((2, N, D))],
    )(x, W1, b1, W2, b2, Wh, bh, Ws, bs, WH0, bH0, WS0, bS0)


def _tc_mid(sumH, sumS, cnt_a, cnt_b, WH1, bH1, WS1, bS1):
    return pl.pallas_call(
        _mid_body,
        grid=_GRID,
        in_specs=[_ROW_SPEC] * 4 + [_W_SPEC, _B_SPEC] * 2,
        out_specs=[_M2_SPEC],
        out_shape=[_F32((2, N, D))],
    )(sumH, sumS, cnt_a, cnt_b, WH1, bH1, WS1, bS1)


def _tc_final(sumH, sumS, cnt_a, cnt_b):
    return pl.pallas_call(
        _fin_body,
        grid=_GRID,
        in_specs=[_ROW_SPEC] * 4,
        out_specs=[_ROW_SPEC] * 2,
        out_shape=[_F32((N, D))] * 2,
    )(sumH, sumS, cnt_a, cnt_b)


_MESH = dict(core_axis_name="c", subcore_axis_name="s")


def _zero_acc(acc_sh, zb_v, row0):
    for j in range(NP // 16 // RZ):
        pltpu.sync_copy(zb_v, acc_sh.at[pl.ds(row0 + j * RZ, RZ)])


def _copy_out(acc_sh, out_hbm, row0, roff):
    for j in range(NP // 16 // RZ):
        r = row0 + j * RZ
        pltpu.sync_copy(acc_sh.at[pl.ds(r, RZ)], out_hbm.at[pl.ds(roff + r, RZ)])


def _sc_agg(m2, src, dst, zeros):
    info = plsc.get_sparse_core_info()
    NS = info.num_subcores
    EPT = E // NS
    NCH = EPT // K

    def body(m2_hbm, src_hbm, dst_hbm, zeros_hbm, out_hbm,
             acc_sh, src_a, dst_a, rows_a, src_b, dst_b, rows_b, zb_v,
             is_a, is_b, gs_a, gs_b, ss_a, ss_b):
        cid = lax.axis_index("c")
        sid = lax.axis_index("s")
        row0 = sid * (NP // NS)

        pltpu.sync_copy(zeros_hbm, zb_v)
        _zero_acc(acc_sh, zb_v, row0)
        plsc.subcore_barrier()

        ebase = sid * EPT

        def iload(j, src_v, dst_v, sem):
            b = ebase + j * K
            pltpu.async_copy(src_hbm.at[pl.ds(cid * E + b, K)], src_v, sem)
            pltpu.async_copy(dst_hbm.at[pl.ds(b, K)], dst_v, sem)

        def iwait(src_v, dst_v, sem):
            pltpu.make_async_copy(src_hbm.at[pl.ds(0, K)], src_v, sem).wait()
            pltpu.make_async_copy(dst_hbm.at[pl.ds(0, K)], dst_v, sem).wait()

        def gather(src_v, rows_v, sem):
            pltpu.async_copy(m2_hbm.at[src_v], rows_v, sem)

        def gwait(rows_v, sem):
            pltpu.make_async_copy(zeros_hbm, rows_v, sem).wait()

        def scat(dst_v, rows_v, sem):
            pltpu.async_copy(rows_v, acc_sh.at[dst_v], sem, add=True)

        def swait(dst_v, rows_v, sem):
            pltpu.make_async_copy(rows_v, acc_sh.at[dst_v], sem).wait()

        iload(0, src_a, dst_a, is_a)
        iload(1, src_b, dst_b, is_b)
        iwait(src_a, dst_a, is_a)
        gather(src_a, rows_a, gs_a)
        iwait(src_b, dst_b, is_b)
        gather(src_b, rows_b, gs_b)

        def mb(jj, _):
            j = 2 * jj
            gwait(rows_a, gs_a)
            scat(dst_a, rows_a, ss_a)
            gwait(rows_b, gs_b)
            scat(dst_b, rows_b, ss_b)
            swait(dst_a, rows_a, ss_a)
            iload(j + 2, src_a, dst_a, is_a)
            swait(dst_b, rows_b, ss_b)
            iload(j + 3, src_b, dst_b, is_b)
            iwait(src_a, dst_a, is_a)
            gather(src_a, rows_a, gs_a)
            iwait(src_b, dst_b, is_b)
            gather(src_b, rows_b, gs_b)
            return 0
        lax.fori_loop(0, NCH // 2 - 1, mb, 0)

        gwait(rows_a, gs_a)
        scat(dst_a, rows_a, ss_a)
        gwait(rows_b, gs_b)
        scat(dst_b, rows_b, ss_b)
        swait(dst_a, rows_a, ss_a)
        swait(dst_b, rows_b, ss_b)

        plsc.subcore_barrier()
        _copy_out(acc_sh, out_hbm, row0, cid * NP)

    call = pl.kernel(
        body,
        out_type=_F32((2 * NP, D)),
        mesh=plsc.VectorSubcoreMesh(**_MESH),
        scratch_types=[
            pltpu.VMEM_SHARED((NP, D), jnp.float32),
            pltpu.VMEM((K,), jnp.int32),
            pltpu.VMEM((K,), jnp.int32),
            pltpu.VMEM((K, D), jnp.float32),
            pltpu.VMEM((K,), jnp.int32),
            pltpu.VMEM((K,), jnp.int32),
            pltpu.VMEM((K, D), jnp.float32),
            pltpu.VMEM((RZ, D), jnp.float32),
            pltpu.SemaphoreType.DMA,
            pltpu.SemaphoreType.DMA,
            pltpu.SemaphoreType.DMA,
            pltpu.SemaphoreType.DMA,
            pltpu.SemaphoreType.DMA,
            pltpu.SemaphoreType.DMA,
        ])
    return call(m2, src, dst, zeros)


def _sc_counts(dst, zeros, ones):
    info = plsc.get_sparse_core_info()
    NS = info.num_subcores
    EPC = E // 2
    EPT = EPC // NS
    NCH = EPT // K

    def body(dst_hbm, zeros_hbm, ones_hbm, out_hbm,
             acc_sh, idx_d, ones_v, zb_v, ss_a, ss_b):
        cid = lax.axis_index("c")
        sid = lax.axis_index("s")
        row0 = sid * (NP // NS)

        pltpu.sync_copy(dst_hbm.at[cid, sid], idx_d)
        pltpu.sync_copy(zeros_hbm, zb_v)
        pltpu.sync_copy(ones_hbm, ones_v)
        _zero_acc(acc_sh, zb_v, row0)
        plsc.subcore_barrier()

        def scat(j, sem):
            pltpu.async_copy(ones_v, acc_sh.at[idx_d.at[j]], sem, add=True)

        def swait(j, sem):
            pltpu.make_async_copy(ones_v, acc_sh.at[idx_d.at[j]], sem).wait()

        scat(0, ss_a)
        scat(1, ss_b)

        def mb(jj, _):
            j = 2 * jj
            swait(j, ss_a)
            scat(j + 2, ss_a)
            swait(j + 1, ss_b)
            scat(j + 3, ss_b)
            return 0
        lax.fori_loop(0, (NCH - 3) // 2, mb, 0)
        swait(NCH - 3, ss_a)
        scat(NCH - 1, ss_a)
        swait(NCH - 2, ss_b)
        swait(NCH - 1, ss_a)
        plsc.subcore_barrier()
        _copy_out(acc_sh, out_hbm, row0, cid * NP)

    call = pl.kernel(
        body,
        out_type=_F32((2 * NP, D)),
        mesh=plsc.VectorSubcoreMesh(**_MESH),
        scratch_types=[
            pltpu.VMEM_SHARED((NP, D), jnp.float32),
            pltpu.VMEM((NCH, K), jnp.int32),
            pltpu.VMEM((K, D), jnp.float32),
            pltpu.VMEM((RZ, D), jnp.float32),
            pltpu.SemaphoreType.DMA,
            pltpu.SemaphoreType.DMA,
        ])
    return call(dst.reshape(2, NS, NCH, K), zeros, ones)


def kernel(x, edge_index, W1, b1, W2, b2, Wh, bh, Ws, bs, WH, bH, WS, bS):
    src = edge_index[0].astype(jnp.int32)
    dst = edge_index[1].astype(jnp.int32)
    src = jnp.concatenate([src, src + N])
    zeros = jnp.zeros((RZ, D), jnp.float32)
    ones = jnp.ones((K, D), jnp.float32)
    r1 = lambda v: v.reshape(1, D)

    cnt = _sc_counts(dst, zeros, ones)
    cnt_a, cnt_b = cnt[:N], cnt[NP:NP + N]

    x_E, m2 = _tc_encoder(
        x, W1, r1(b1), W2, r1(b2), Wh, r1(bh), Ws, r1(bs),
        WH[0], r1(bH[0]), WS[0], r1(bS[0]))

    sums = _sc_agg(m2.reshape(2 * N, D), src, dst, zeros)
    (m2,) = _tc_mid(sums[:N], sums[NP:NP + N], cnt_a, cnt_b,
                    WH[1], r1(bH[1]), WS[1], r1(bS[1]))

    sums2 = _sc_agg(m2.reshape(2 * N, D), src, dst, zeros)
    x_H, x_S = _tc_final(sums2[:N], sums2[NP:NP + N], cnt_a, cnt_b)
    return (x_E, x_H, x_S)
